# baseline (speedup 1.0000x reference)
import functools
import math

import jax
import jax.numpy as jnp
from jax.experimental import pallas as pl
from jax.experimental.pallas import tpu as pltpu

F32 = jnp.float32
BF16 = jnp.bfloat16
_DNUM = (((1,), (1,)), ((), ()))


def _ln(x, g, b):
    m = jnp.mean(x, axis=-1, keepdims=True)
    v = jnp.mean((x - m) ** 2, axis=-1, keepdims=True)
    return (x - m) * jax.lax.rsqrt(v + 1e-5) * g + b


def _gelu(x):
    return 0.5 * x * (1.0 + jax.lax.erf(x * (1.0 / math.sqrt(2.0))))


def _dotn(a, w):
    return jax.lax.dot_general(a.astype(BF16), w.astype(BF16), _DNUM,
                               preferred_element_type=F32)


def _softmax_ctx(q, k, v, scale):
    qs = (q * scale).astype(BF16)
    s = jax.lax.dot_general(qs, k, _DNUM, preferred_element_type=F32)
    m = jnp.max(s, axis=-1, keepdims=True)
    e = jnp.exp((s - m).astype(BF16))
    r = jax.lax.reciprocal(jnp.sum(e.astype(F32), axis=-1, keepdims=True))
    p = e * r.astype(BF16)
    return jax.lax.dot_general(p, v, (((1,), (0,)), ((), ())),
                               preferred_element_type=F32)


def _body(x_ref, pe_ref, wp_ref, bp_ref,
          wqkv_ref, bqkv_ref, wo_ref, bo_ref, g1_ref, b1n_ref,
          w1_ref, bf1_ref, w2_ref, bf2_ref, g2_ref, b2n_ref,
          wkv_ref, bkv_ref, wq_ref, bq_ref,
          wo2_ref, bo2_ref, g12_ref, b12_ref,
          w12_ref, bf12_ref, w22_ref, bf22_ref, g22_ref, b22_ref,
          fg_ref, fb_ref, gw_ref, gb_ref,
          ew1_ref, eb1_ref, ew2_ref, eb2_ref,
          mg_ref, mb_ref, wd_ref, bd_ref, xl_ref,
          o_ref, qkv_s, ffh_s, kv_s,
          h, dh, scale, n_exp, ffc):
    pe = pe_ref[...]
    b_, t_, f_ = x_ref.shape
    d = wp_ref.shape[0]

    resids = []
    for b in range(b_):
        x = x_ref[b]
        acc = x
        for off in (1, 2, 3):
            zpad = jnp.zeros((off, f_), x.dtype)
            acc = acc + jnp.concatenate([zpad, x[: t_ - off]], axis=0)
            acc = acc + jnp.concatenate([x[off:], zpad], axis=0)
        resids.append(x - acc * (1.0 / 7.0))
    resid = jnp.concatenate(resids, axis=0)
    pe4 = jnp.concatenate([pe] * b_, axis=0)
    z0 = _dotn(resid, wp_ref[...]) + bp_ref[...] + pe4

    qkv_s[...] = (_dotn(z0, wqkv_ref[...]) + bqkv_ref[...]).astype(BF16)
    ctxs = []
    for b in range(b_):
        rows = slice(b * t_, (b + 1) * t_)
        parts = []
        for hh in range(h):
            q = qkv_s[rows, hh * dh:(hh + 1) * dh]
            k = qkv_s[rows, d + hh * dh:d + (hh + 1) * dh]
            v = qkv_s[rows, 2 * d + hh * dh:2 * d + (hh + 1) * dh]
            parts.append(_softmax_ctx(q, k, v, scale))
        ctxs.append(jnp.concatenate(parts, axis=1))
    ctx = jnp.concatenate(ctxs, axis=0)

    z1 = _ln(z0 + _dotn(ctx, wo_ref[...]) + bo_ref[...],
             g1_ref[...], b1n_ref[...])

    ffsum = None
    for c in range(ffc):
        cw = (4 * d) // ffc
        sl = slice(c * cw, (c + 1) * cw)
        ffh_s[...] = _gelu(
            (_dotn(z1, w1_ref[sl, :]) + bf1_ref[:, sl]).astype(BF16))
        part = _dotn(ffh_s[...], w2_ref[:, sl])
        ffsum = part if ffsum is None else ffsum + part
    z2 = _ln(z1 + ffsum + bf2_ref[...], g2_ref[...], b2n_ref[...])

    kv_s[...] = (_dotn(z2, wkv_ref[...]) + bkv_ref[...]).astype(BF16)
    zls, cls = [], []
    for b in range(b_):
        rows = slice(b * t_, (b + 1) * t_)
        zlb = z2[b * t_ + t_ - 1:(b + 1) * t_, :]
        qlast = _dotn(zlb, wq_ref[...]) + bq_ref[...]
        parts = []
        for hh in range(h):
            qh = qlast[:, hh * dh:(hh + 1) * dh]
            k = kv_s[rows, hh * dh:(hh + 1) * dh]
            v = kv_s[rows, d + hh * dh:d + (hh + 1) * dh]
            parts.append(_softmax_ctx(qh, k, v, scale))
        zls.append(zlb)
        cls.append(jnp.concatenate(parts, axis=1))
    zl = jnp.concatenate(zls, axis=0)
    ctxl = jnp.concatenate(cls, axis=0)

    z1t = _ln(zl + _dotn(ctxl, wo2_ref[...]) + bo2_ref[...],
              g12_ref[...], b12_ref[...])
    ffht = _gelu(_dotn(z1t, w12_ref[...]) + bf12_ref[...])
    z2t = _ln(z1t + _dotn(ffht, w22_ref[...]) + bf22_ref[...],
              g22_ref[...], b22_ref[...])
    zf = _ln(z2t, fg_ref[...], fb_ref[...])

    logits = jax.lax.dot_general(zf, gw_ref[...], _DNUM,
                                 preferred_element_type=F32) + gb_ref[...]
    lcols = [logits[:, e:e + 1] for e in range(n_exp)]
    m1 = lcols[0]
    for le in lcols[1:]:
        m1 = jnp.maximum(m1, le)
    first, taken = [], None
    for le in lcols:
        is_e = (le == m1) if taken is None else jnp.logical_and(
            le == m1, jnp.logical_not(taken))
        first.append(is_e)
        taken = is_e if taken is None else jnp.logical_or(taken, is_e)
    masked = [jnp.where(f, -1e30, le) for f, le in zip(first, lcols)]
    m2 = masked[0]
    for le in masked[1:]:
        m2 = jnp.maximum(m2, le)
    second, taken2 = [], None
    for le in masked:
        is_e = (le == m2) if taken2 is None else jnp.logical_and(
            le == m2, jnp.logical_not(taken2))
        second.append(is_e)
        taken2 = is_e if taken2 is None else jnp.logical_or(taken2, is_e)
    w1c = 1.0 / (1.0 + jnp.exp(m2 - m1))
    mixed = jnp.zeros_like(zf)
    for e in range(n_exp):
        he = jnp.maximum(_dotn(zf, ew1_ref[e]) + eb1_ref[e], 0.0)
        oe = _dotn(he, ew2_ref[e]) + eb2_ref[e]
        coeff_e = w1c * first[e].astype(F32) + (1.0 - w1c) * second[e].astype(F32)
        mixed = mixed + coeff_e * oe

    zm = _ln(mixed + zf, mg_ref[...], mb_ref[...])
    xl = xl_ref[...]
    trend_last = (xl[:, 0] + xl[:, 1] + xl[:, 2] + xl[:, 3]) * (1.0 / 7.0)
    nr = wd_ref.shape[0]
    o_ref[...] = _dotn(zm, wd_ref[...]) + bd_ref[...] + trend_last[:, :nr]


def kernel(x_series, Wp, bp, pos_enc, Wqkv, bqkv, Wo, bo, ln1_g, ln1_b,
           ln2_g, ln2_b, W1, b1, W2, b2, fin_g, fin_b, gate_W, gate_b,
           expW1, expb1, expW2, expb2, moe_g, moe_b, Wd, bd):
    b_, t_, f_ = x_series.shape
    d = Wp.shape[0]
    h = 8
    dh = d // h
    n_exp = gate_W.shape[0]
    nr = Wd.shape[0]
    scale = 1.0 / math.sqrt(float(dh))
    ffc = 4

    def bcast2d(v):
        return v.reshape(1, v.shape[-1])

    def fullspec(shape):
        nd = len(shape)
        return pl.BlockSpec(shape, lambda *a, _nd=nd: (0,) * _nd)

    x_last4 = x_series[:, t_ - 4:, :]
    args = (
        x_series, pos_enc[:t_], Wp, bcast2d(bp),
        Wqkv[0], bcast2d(bqkv[0]), Wo[0], bcast2d(bo[0]),
        bcast2d(ln1_g[0]), bcast2d(ln1_b[0]),
        W1[0], bcast2d(b1[0]), W2[0], bcast2d(b2[0]),
        bcast2d(ln2_g[0]), bcast2d(ln2_b[0]),
        Wqkv[1, d:], bcast2d(bqkv[1, d:]),
        Wqkv[1, :d], bcast2d(bqkv[1, :d]),
        Wo[1], bcast2d(bo[1]),
        bcast2d(ln1_g[1]), bcast2d(ln1_b[1]),
        W1[1], bcast2d(b1[1]), W2[1], bcast2d(b2[1]),
        bcast2d(ln2_g[1]), bcast2d(ln2_b[1]),
        bcast2d(fin_g), bcast2d(fin_b),
        gate_W, bcast2d(gate_b),
        expW1.astype(BF16), expb1.reshape(n_exp, 1, 2 * d),
        expW2.astype(BF16), expb2.reshape(n_exp, 1, d),
        bcast2d(moe_g), bcast2d(moe_b),
        Wd, bcast2d(bd), x_last4,
    )
    out = pl.pallas_call(
        functools.partial(_body, h=h, dh=dh, scale=scale, n_exp=n_exp,
                          ffc=ffc),
        in_specs=[fullspec(a.shape) for a in args],
        out_specs=fullspec((b_, nr)),
        out_shape=jax.ShapeDtypeStruct((b_, nr), F32),
        scratch_shapes=[
            pltpu.VMEM((b_ * t_, 3 * d), BF16),
            pltpu.VMEM((b_ * t_, (4 * d) // ffc), BF16),
            pltpu.VMEM((b_ * t_, 2 * d), BF16),
        ],
        compiler_params=pltpu.CompilerParams(
            vmem_limit_bytes=63 * 1024 * 1024),
    )(*args)
    return out


# speedup vs baseline: 1.4688x; 1.4688x over previous
import functools
import math

import jax
import jax.numpy as jnp
from jax.experimental import pallas as pl
from jax.experimental.pallas import tpu as pltpu

F32 = jnp.float32
BF16 = jnp.bfloat16
_DNUM = (((1,), (1,)), ((), ()))


def _ln(x, g, b):
    m = jnp.mean(x, axis=-1, keepdims=True)
    v = jnp.mean((x - m) ** 2, axis=-1, keepdims=True)
    return (x - m) * jax.lax.rsqrt(v + 1e-5) * g + b


def _gelu(x):
    return 0.5 * x * (1.0 + jax.lax.erf(x * (1.0 / math.sqrt(2.0))))


def _dotn(a, w):
    return jax.lax.dot_general(a.astype(BF16), w.astype(BF16), _DNUM,
                               preferred_element_type=F32)


def _softmax_ctx(q, k, v, scale):
    qs = (q * scale).astype(BF16)
    s = jax.lax.dot_general(qs, k, _DNUM, preferred_element_type=F32)
    m = jnp.max(s, axis=-1, keepdims=True)
    e = jnp.exp((s - m).astype(BF16))
    r = jax.lax.reciprocal(jnp.sum(e.astype(F32), axis=-1, keepdims=True))
    p = e * r.astype(BF16)
    return jax.lax.dot_general(p, v, (((1,), (0,)), ((), ())),
                               preferred_element_type=F32)


def _body_a(x_ref, pe_ref, wp_ref, bp_ref,
            wqkv_ref, bqkv_ref, wo_ref, bo_ref, g1_ref, b1n_ref,
            w1_ref, bf1_ref, w2_ref, bf2_ref, g2_ref, b2n_ref,
            zl_ref, cl_ref,
            wp_s, wqkv_s, wo_s, w1_s, w2_s, wkv_s, wq_s, qkv_s, ffh_s, kv_s,
            h, dh, scale):
    dd = wp_ref.shape[0]
    wp_s[...] = wp_ref[...].astype(BF16)
    wqkv_s[...] = wqkv_ref[0].astype(BF16)
    wo_s[...] = wo_ref[0].astype(BF16)
    w1_s[...] = w1_ref[0].astype(BF16)
    w2_s[...] = w2_ref[0].astype(BF16)
    wkv_s[...] = wqkv_ref[1, dd:, :].astype(BF16)
    wq_s[...] = wqkv_ref[1, :dd, :].astype(BF16)
    bqkv2 = bqkv_ref[1:2, :]
    bkv = bqkv2[:, dd:]
    bq = bqkv2[:, :dd]
    pe = pe_ref[...]
    b_, t_, f_ = x_ref.shape
    d = wp_s.shape[0]

    resids = []
    for b in range(b_):
        x = x_ref[b]
        acc = x
        for off in (1, 2, 3):
            zpad = jnp.zeros((off, f_), x.dtype)
            acc = acc + jnp.concatenate([zpad, x[: t_ - off]], axis=0)
            acc = acc + jnp.concatenate([x[off:], zpad], axis=0)
        resids.append(x - acc * (1.0 / 7.0))
    resid = jnp.concatenate(resids, axis=0)
    pe4 = jnp.concatenate([pe] * b_, axis=0)
    z0 = _dotn(resid, wp_s[...]) + bp_ref[...] + pe4

    qkv_s[...] = (_dotn(z0, wqkv_s[...]) + bqkv_ref[0:1, :]).astype(BF16)
    ctxs = []
    for b in range(b_):
        rows = slice(b * t_, (b + 1) * t_)
        parts = []
        for hh in range(h):
            q = qkv_s[rows, hh * dh:(hh + 1) * dh]
            k = qkv_s[rows, d + hh * dh:d + (hh + 1) * dh]
            v = qkv_s[rows, 2 * d + hh * dh:2 * d + (hh + 1) * dh]
            parts.append(_softmax_ctx(q, k, v, scale))
        ctxs.append(jnp.concatenate(parts, axis=1))
    ctx = jnp.concatenate(ctxs, axis=0)

    z1 = _ln(z0 + _dotn(ctx, wo_s[...]) + bo_ref[0:1, :],
             g1_ref[0:1, :], b1n_ref[0:1, :])
    ffh_s[...] = _gelu((_dotn(z1, w1_s[...]) + bf1_ref[0:1, :]).astype(BF16))
    z2 = _ln(z1 + _dotn(ffh_s[...], w2_s[...]) + bf2_ref[0:1, :],
             g2_ref[0:1, :], b2n_ref[0:1, :])

    kv_s[...] = (_dotn(z2, wkv_s[...]) + bkv).astype(BF16)
    zls, cls = [], []
    for b in range(b_):
        rows = slice(b * t_, (b + 1) * t_)
        zl = z2[b * t_ + t_ - 1:(b + 1) * t_, :]
        qlast = _dotn(zl, wq_s[...]) + bq
        parts = []
        for hh in range(h):
            qh = qlast[:, hh * dh:(hh + 1) * dh]
            k = kv_s[rows, hh * dh:(hh + 1) * dh]
            v = kv_s[rows, d + hh * dh:d + (hh + 1) * dh]
            parts.append(_softmax_ctx(qh, k, v, scale))
        zls.append(zl)
        cls.append(jnp.concatenate(parts, axis=1))
    zl_ref[...] = jnp.concatenate(zls, axis=0)
    cl_ref[...] = jnp.concatenate(cls, axis=0)


def _body_tail(zl_ref, ctx_ref, wo_ref, bo_ref, g1_ref, b1n_ref,
               w1_ref, bf1_ref, w2_ref, bf2_ref, g2_ref, b2n_ref,
               fg_ref, fb_ref, gw_ref, gb_ref,
               ew1_ref, eb1_ref, ew2_ref, eb2_ref,
               mg_ref, mb_ref, wd_ref, bd_ref, xl_ref, o_ref, n_exp):
    zl = zl_ref[...]
    ctx = ctx_ref[...]
    z1 = _ln(zl + _dotn(ctx, wo_ref[0]) + bo_ref[1:2, :],
             g1_ref[1:2, :], b1n_ref[1:2, :])
    ffh = _gelu(_dotn(z1, w1_ref[0]) + bf1_ref[1:2, :])
    z2 = _ln(z1 + _dotn(ffh, w2_ref[0]) + bf2_ref[1:2, :],
             g2_ref[1:2, :], b2n_ref[1:2, :])
    zf = _ln(z2, fg_ref[...], fb_ref[...])

    logits = jax.lax.dot_general(zf, gw_ref[...], _DNUM,
                                 preferred_element_type=F32) + gb_ref[...]
    lcols = [logits[:, e:e + 1] for e in range(n_exp)]
    m1 = lcols[0]
    for le in lcols[1:]:
        m1 = jnp.maximum(m1, le)
    first, taken = [], None
    for le in lcols:
        is_e = (le == m1) if taken is None else jnp.logical_and(
            le == m1, jnp.logical_not(taken))
        first.append(is_e)
        taken = is_e if taken is None else jnp.logical_or(taken, is_e)
    masked = [jnp.where(f, -1e30, le) for f, le in zip(first, lcols)]
    m2 = masked[0]
    for le in masked[1:]:
        m2 = jnp.maximum(m2, le)
    second, taken2 = [], None
    for le in masked:
        is_e = (le == m2) if taken2 is None else jnp.logical_and(
            le == m2, jnp.logical_not(taken2))
        second.append(is_e)
        taken2 = is_e if taken2 is None else jnp.logical_or(taken2, is_e)
    w1c = 1.0 / (1.0 + jnp.exp(m2 - m1))
    mixed = jnp.zeros_like(zf)
    for e in range(n_exp):
        he = jnp.maximum(_dotn(zf, ew1_ref[e]) + eb1_ref[e], 0.0)
        oe = _dotn(he, ew2_ref[e]) + eb2_ref[e]
        coeff_e = w1c * first[e].astype(F32) + (1.0 - w1c) * second[e].astype(F32)
        mixed = mixed + coeff_e * oe

    zm = _ln(mixed + zf, mg_ref[...], mb_ref[...])
    xl = xl_ref[...]
    trend_last = (xl[:, 0] + xl[:, 1] + xl[:, 2] + xl[:, 3]) * (1.0 / 7.0)
    nr = wd_ref.shape[0]
    o_ref[...] = _dotn(zm, wd_ref[...]) + bd_ref[...] + trend_last[:, :nr]


def kernel(x_series, Wp, bp, pos_enc, Wqkv, bqkv, Wo, bo, ln1_g, ln1_b,
           ln2_g, ln2_b, W1, b1, W2, b2, fin_g, fin_b, gate_W, gate_b,
           expW1, expb1, expW2, expb2, moe_g, moe_b, Wd, bd):
    b_, t_, f_ = x_series.shape
    d = Wp.shape[0]
    h = 8
    dh = d // h
    n_exp = gate_W.shape[0]
    nr = Wd.shape[0]
    scale = 1.0 / math.sqrt(float(dh))

    def bcast2d(v):
        return v.reshape(1, v.shape[-1])

    def fullspec(shape):
        nd = len(shape)
        return pl.BlockSpec(shape, lambda *a, _nd=nd: (0,) * _nd)

    def layerspec(shape, l):
        blk = (1,) + shape[1:]
        return pl.BlockSpec(blk, lambda *a, _l=l: (_l, 0, 0))

    args_a = (
        x_series, pos_enc[:t_], Wp, bcast2d(bp),
        Wqkv, bqkv, Wo, bo, ln1_g, ln1_b,
        W1, b1, W2, b2, ln2_g, ln2_b,
    )
    specs_a = [fullspec(a.shape) for a in args_a]
    specs_a[6] = layerspec(Wo.shape, 0)
    specs_a[10] = layerspec(W1.shape, 0)
    specs_a[12] = layerspec(W2.shape, 0)
    z_last, ctx_last = pl.pallas_call(
        functools.partial(_body_a, h=h, dh=dh, scale=scale),
        grid=(1,),
        in_specs=specs_a,
        out_specs=[fullspec((b_, d)), fullspec((b_, d))],
        out_shape=[jax.ShapeDtypeStruct((b_, d), F32),
                   jax.ShapeDtypeStruct((b_, d), F32)],
        scratch_shapes=[
            pltpu.VMEM((d, f_), BF16),
            pltpu.VMEM((3 * d, d), BF16),
            pltpu.VMEM((d, d), BF16),
            pltpu.VMEM((4 * d, d), BF16),
            pltpu.VMEM((d, 4 * d), BF16),
            pltpu.VMEM((2 * d, d), BF16),
            pltpu.VMEM((d, d), BF16),
            pltpu.VMEM((b_ * t_, 3 * d), BF16),
            pltpu.VMEM((b_ * t_, 4 * d), BF16),
            pltpu.VMEM((b_ * t_, 2 * d), BF16),
        ],
        compiler_params=pltpu.CompilerParams(
            vmem_limit_bytes=60 * 1024 * 1024),
    )(*args_a)

    x_last4 = x_series[:, t_ - 4:, :]
    args_t = (
        z_last, ctx_last, Wo, bo, ln1_g, ln1_b,
        W1, b1, W2, b2, ln2_g, ln2_b,
        bcast2d(fin_g), bcast2d(fin_b),
        gate_W, bcast2d(gate_b),
        expW1, expb1.reshape(n_exp, 1, 2 * d),
        expW2, expb2.reshape(n_exp, 1, d),
        bcast2d(moe_g), bcast2d(moe_b),
        Wd, bcast2d(bd), x_last4,
    )
    specs_t = [fullspec(a.shape) for a in args_t]
    specs_t[2] = layerspec(Wo.shape, 1)
    specs_t[6] = layerspec(W1.shape, 1)
    specs_t[8] = layerspec(W2.shape, 1)
    out = pl.pallas_call(
        functools.partial(_body_tail, n_exp=n_exp),
        grid=(1,),
        in_specs=specs_t,
        out_specs=fullspec((b_, nr)),
        out_shape=jax.ShapeDtypeStruct((b_, nr), F32),
        compiler_params=pltpu.CompilerParams(
            vmem_limit_bytes=60 * 1024 * 1024),
    )(*args_t)
    return out


# speedup vs baseline: 1.4923x; 1.0160x over previous
import functools
import math

import jax
import jax.numpy as jnp
from jax.experimental import pallas as pl
from jax.experimental.pallas import tpu as pltpu

F32 = jnp.float32
BF16 = jnp.bfloat16
_DNUM = (((1,), (1,)), ((), ()))


def _ln(x, g, b):
    m = jnp.mean(x, axis=-1, keepdims=True)
    v = jnp.mean((x - m) ** 2, axis=-1, keepdims=True)
    return (x - m) * jax.lax.rsqrt(v + 1e-5) * g + b


def _gelu(x):
    return 0.5 * x * (1.0 + jax.lax.erf(x * (1.0 / math.sqrt(2.0))))


def _dotn(a, w):
    return jax.lax.dot_general(a.astype(BF16), w.astype(BF16), _DNUM,
                               preferred_element_type=F32)


def _softmax_ctx(q, k, v, scale):
    qs = (q * scale).astype(BF16)
    s = jax.lax.dot_general(qs, k, _DNUM, preferred_element_type=F32)
    m = jnp.max(s, axis=-1, keepdims=True)
    e = jnp.exp((s - m).astype(BF16))
    r = jax.lax.reciprocal(jnp.sum(e.astype(F32), axis=-1, keepdims=True))
    u = jax.lax.dot_general(e, v, (((1,), (0,)), ((), ())),
                            preferred_element_type=F32)
    return u * r


def _body_a(x_ref, pe_ref, wp_ref, bp_ref,
            wqkv_ref, bqkv_ref, wo_ref, bo_ref, g1_ref, b1n_ref,
            w1_ref, bf1_ref, w2_ref, bf2_ref, g2_ref, b2n_ref,
            zl_ref, cl_ref,
            wp_s, wqkv_s, wo_s, w1_s, w2_s, wkv_s, wq_s, qkv_s, ffh_s, kv_s,
            h, dh, scale):
    dd = wp_ref.shape[0]
    wp_s[...] = wp_ref[...].astype(BF16)
    wqkv_s[...] = wqkv_ref[0].astype(BF16)
    wo_s[...] = wo_ref[0].astype(BF16)
    w1_s[...] = w1_ref[0].astype(BF16)
    w2_s[...] = w2_ref[0].astype(BF16)
    wkv_s[...] = wqkv_ref[1, dd:, :].astype(BF16)
    wq_s[...] = wqkv_ref[1, :dd, :].astype(BF16)
    bqkv2 = bqkv_ref[1:2, :]
    bkv = bqkv2[:, dd:]
    bq = bqkv2[:, :dd]
    pe = pe_ref[...]
    b_, t_, f_ = x_ref.shape
    d = wp_s.shape[0]

    resids = []
    for b in range(b_):
        x = x_ref[b]
        acc = x
        for off in (1, 2, 3):
            zpad = jnp.zeros((off, f_), x.dtype)
            acc = acc + jnp.concatenate([zpad, x[: t_ - off]], axis=0)
            acc = acc + jnp.concatenate([x[off:], zpad], axis=0)
        resids.append(x - acc * (1.0 / 7.0))
    resid = jnp.concatenate(resids, axis=0)
    pe4 = jnp.concatenate([pe] * b_, axis=0)
    z0 = _dotn(resid, wp_s[...]) + bp_ref[...] + pe4

    qkv_s[...] = (_dotn(z0, wqkv_s[...]) + bqkv_ref[0:1, :]).astype(BF16)
    ctxs = []
    for b in range(b_):
        rows = slice(b * t_, (b + 1) * t_)
        parts = []
        for hh in range(h):
            q = qkv_s[rows, hh * dh:(hh + 1) * dh]
            k = qkv_s[rows, d + hh * dh:d + (hh + 1) * dh]
            v = qkv_s[rows, 2 * d + hh * dh:2 * d + (hh + 1) * dh]
            parts.append(_softmax_ctx(q, k, v, scale))
        ctxs.append(jnp.concatenate(parts, axis=1))
    ctx = jnp.concatenate(ctxs, axis=0)

    z1 = _ln(z0 + _dotn(ctx, wo_s[...]) + bo_ref[0:1, :],
             g1_ref[0:1, :], b1n_ref[0:1, :])
    ffh_s[...] = _gelu((_dotn(z1, w1_s[...]) + bf1_ref[0:1, :]).astype(BF16))
    z2 = _ln(z1 + _dotn(ffh_s[...], w2_s[...]) + bf2_ref[0:1, :],
             g2_ref[0:1, :], b2n_ref[0:1, :])

    kv_s[...] = (_dotn(z2, wkv_s[...]) + bkv).astype(BF16)
    zls, cls = [], []
    for b in range(b_):
        rows = slice(b * t_, (b + 1) * t_)
        zl = z2[b * t_ + t_ - 1:(b + 1) * t_, :]
        qlast = _dotn(zl, wq_s[...]) + bq
        parts = []
        for hh in range(h):
            qh = qlast[:, hh * dh:(hh + 1) * dh]
            k = kv_s[rows, hh * dh:(hh + 1) * dh]
            v = kv_s[rows, d + hh * dh:d + (hh + 1) * dh]
            parts.append(_softmax_ctx(qh, k, v, scale))
        zls.append(zl)
        cls.append(jnp.concatenate(parts, axis=1))
    zl_ref[...] = jnp.concatenate(zls, axis=0)
    cl_ref[...] = jnp.concatenate(cls, axis=0)


def _body_tail(zl_ref, ctx_ref, wo_ref, bo_ref, g1_ref, b1n_ref,
               w1_ref, bf1_ref, w2_ref, bf2_ref, g2_ref, b2n_ref,
               fg_ref, fb_ref, gw_ref, gb_ref,
               ew1_ref, eb1_ref, ew2_ref, eb2_ref,
               mg_ref, mb_ref, wd_ref, bd_ref, xl_ref, o_ref, n_exp):
    zl = zl_ref[...]
    ctx = ctx_ref[...]
    z1 = _ln(zl + _dotn(ctx, wo_ref[0]) + bo_ref[1:2, :],
             g1_ref[1:2, :], b1n_ref[1:2, :])
    ffh = _gelu(_dotn(z1, w1_ref[0]) + bf1_ref[1:2, :])
    z2 = _ln(z1 + _dotn(ffh, w2_ref[0]) + bf2_ref[1:2, :],
             g2_ref[1:2, :], b2n_ref[1:2, :])
    zf = _ln(z2, fg_ref[...], fb_ref[...])

    logits = jax.lax.dot_general(zf, gw_ref[...], _DNUM,
                                 preferred_element_type=F32) + gb_ref[...]
    lcols = [logits[:, e:e + 1] for e in range(n_exp)]
    m1 = lcols[0]
    for le in lcols[1:]:
        m1 = jnp.maximum(m1, le)
    first, taken = [], None
    for le in lcols:
        is_e = (le == m1) if taken is None else jnp.logical_and(
            le == m1, jnp.logical_not(taken))
        first.append(is_e)
        taken = is_e if taken is None else jnp.logical_or(taken, is_e)
    masked = [jnp.where(f, -1e30, le) for f, le in zip(first, lcols)]
    m2 = masked[0]
    for le in masked[1:]:
        m2 = jnp.maximum(m2, le)
    second, taken2 = [], None
    for le in masked:
        is_e = (le == m2) if taken2 is None else jnp.logical_and(
            le == m2, jnp.logical_not(taken2))
        second.append(is_e)
        taken2 = is_e if taken2 is None else jnp.logical_or(taken2, is_e)
    w1c = 1.0 / (1.0 + jnp.exp(m2 - m1))
    mixed = jnp.zeros_like(zf)
    for e in range(n_exp):
        he = jnp.maximum(_dotn(zf, ew1_ref[e]) + eb1_ref[e], 0.0)
        oe = _dotn(he, ew2_ref[e]) + eb2_ref[e]
        coeff_e = w1c * first[e].astype(F32) + (1.0 - w1c) * second[e].astype(F32)
        mixed = mixed + coeff_e * oe

    zm = _ln(mixed + zf, mg_ref[...], mb_ref[...])
    xl = xl_ref[...]
    trend_last = (xl[:, 0] + xl[:, 1] + xl[:, 2] + xl[:, 3]) * (1.0 / 7.0)
    nr = wd_ref.shape[0]
    o_ref[...] = _dotn(zm, wd_ref[...]) + bd_ref[...] + trend_last[:, :nr]


def kernel(x_series, Wp, bp, pos_enc, Wqkv, bqkv, Wo, bo, ln1_g, ln1_b,
           ln2_g, ln2_b, W1, b1, W2, b2, fin_g, fin_b, gate_W, gate_b,
           expW1, expb1, expW2, expb2, moe_g, moe_b, Wd, bd):
    b_, t_, f_ = x_series.shape
    d = Wp.shape[0]
    h = 8
    dh = d // h
    n_exp = gate_W.shape[0]
    nr = Wd.shape[0]
    scale = 1.0 / math.sqrt(float(dh))

    def bcast2d(v):
        return v.reshape(1, v.shape[-1])

    def fullspec(shape):
        nd = len(shape)
        return pl.BlockSpec(shape, lambda *a, _nd=nd: (0,) * _nd)

    def layerspec(shape, l):
        blk = (1,) + shape[1:]
        return pl.BlockSpec(blk, lambda *a, _l=l: (_l, 0, 0))

    args_a = (
        x_series, pos_enc[:t_], Wp, bcast2d(bp),
        Wqkv, bqkv, Wo, bo, ln1_g, ln1_b,
        W1, b1, W2, b2, ln2_g, ln2_b,
    )
    specs_a = [fullspec(a.shape) for a in args_a]
    specs_a[6] = layerspec(Wo.shape, 0)
    specs_a[10] = layerspec(W1.shape, 0)
    specs_a[12] = layerspec(W2.shape, 0)
    z_last, ctx_last = pl.pallas_call(
        functools.partial(_body_a, h=h, dh=dh, scale=scale),
        grid=(1,),
        in_specs=specs_a,
        out_specs=[fullspec((b_, d)), fullspec((b_, d))],
        out_shape=[jax.ShapeDtypeStruct((b_, d), F32),
                   jax.ShapeDtypeStruct((b_, d), F32)],
        scratch_shapes=[
            pltpu.VMEM((d, f_), BF16),
            pltpu.VMEM((3 * d, d), BF16),
            pltpu.VMEM((d, d), BF16),
            pltpu.VMEM((4 * d, d), BF16),
            pltpu.VMEM((d, 4 * d), BF16),
            pltpu.VMEM((2 * d, d), BF16),
            pltpu.VMEM((d, d), BF16),
            pltpu.VMEM((b_ * t_, 3 * d), BF16),
            pltpu.VMEM((b_ * t_, 4 * d), BF16),
            pltpu.VMEM((b_ * t_, 2 * d), BF16),
        ],
        compiler_params=pltpu.CompilerParams(
            vmem_limit_bytes=60 * 1024 * 1024),
    )(*args_a)

    x_last4 = x_series[:, t_ - 4:, :]
    args_t = (
        z_last, ctx_last, Wo, bo, ln1_g, ln1_b,
        W1, b1, W2, b2, ln2_g, ln2_b,
        bcast2d(fin_g), bcast2d(fin_b),
        gate_W, bcast2d(gate_b),
        expW1, expb1.reshape(n_exp, 1, 2 * d),
        expW2, expb2.reshape(n_exp, 1, d),
        bcast2d(moe_g), bcast2d(moe_b),
        Wd, bcast2d(bd), x_last4,
    )
    specs_t = [fullspec(a.shape) for a in args_t]
    specs_t[2] = layerspec(Wo.shape, 1)
    specs_t[6] = layerspec(W1.shape, 1)
    specs_t[8] = layerspec(W2.shape, 1)
    out = pl.pallas_call(
        functools.partial(_body_tail, n_exp=n_exp),
        grid=(1,),
        in_specs=specs_t,
        out_specs=fullspec((b_, nr)),
        out_shape=jax.ShapeDtypeStruct((b_, nr), F32),
        compiler_params=pltpu.CompilerParams(
            vmem_limit_bytes=60 * 1024 * 1024),
    )(*args_t)
    return out
